# baseline (speedup 1.0000x reference)
import functools

import jax
import jax.numpy as jnp
from jax import lax
from jax.experimental import pallas as pl
from jax.experimental.pallas import tpu as pltpu
from jax.experimental.pallas import tpu_sc as plsc

_N = 10000
_E = 320000
_U = 128
_REPS = 5

_NC = 2
_NS = 16
_NW = _NC * _NS
_CH = 64
_NCHUNK = 160
_EPW = _NCHUNK * _CH
_EPAD = _NW * _EPW
_NBUF = 5
_NIDX = 10
_GRP = 10
_RPS = 624
_RBLK = 16


def _sc_body(state, cre, acc, gbuf, idxb, acc_sh, semg, sems, semi):
    c = lax.axis_index("c")
    s = lax.axis_index("s")
    wid = s * _NC + c

    rbase = s * _RPS
    nblk = jnp.where(s == _NS - 1, (_N - (_NS - 1) * _RPS) // _RBLK,
                     _RPS // _RBLK)

    zv = jnp.zeros((16,), jnp.float32)
    for rr in range(_RBLK):
        for j in range(_U // 16):
            gbuf[0, rr, pl.ds(j * 16, 16)] = zv

    def _zero(k, carry):
        pltpu.sync_copy(gbuf.at[0, pl.ds(0, _RBLK)],
                        acc_sh.at[pl.ds(rbase + k * _RBLK, _RBLK)])
        return carry

    lax.fori_loop(0, nblk, _zero, 0)
    plsc.subcore_barrier()

    def _step(i, m):
        b = m % _NBUF
        pltpu.make_async_copy(state.at[pl.ds(0, _CH)], gbuf.at[b],
                              semg.at[b]).wait()

        def _scale(q, c2, b=b, m=m):
            ev16 = lax.bitcast_convert_type(
                idxb[m, 2, pl.ds(q * 16, 16)], jnp.float32)
            for l in range(16):
                sval = ev16[l]
                for j in range(_U // 16):
                    gbuf[b, q * 16 + l, pl.ds(j * 16, 16)] = (
                        gbuf[b, q * 16 + l, pl.ds(j * 16, 16)] * sval)
            return c2

        lax.fori_loop(0, _CH // 16, _scale, 0)

        pltpu.async_copy(gbuf.at[b], acc_sh.at[idxb.at[m, 1]], sems.at[b],
                         add=True)

        b3 = (b + 3) % _NBUF

        @pl.when(i >= 2)
        def _():
            pltpu.make_async_copy(gbuf.at[b3], acc_sh.at[pl.ds(0, _CH)],
                                  sems.at[b3]).wait()

        m3 = (m + 3) % _NIDX

        @pl.when(i + 3 < _NCHUNK)
        def _():
            pltpu.make_async_copy(cre.at[wid, 0], idxb.at[m3],
                                  semi.at[m3]).wait()
            pltpu.async_copy(state.at[idxb.at[m3, 0]], gbuf.at[b3],
                             semg.at[b3])

        m7 = (m + 7) % _NIDX

        @pl.when(i + 7 < _NCHUNK)
        def _():
            pltpu.async_copy(cre.at[wid, i + 7], idxb.at[m7], semi.at[m7])

    for j in range(3):
        pltpu.sync_copy(cre.at[wid, j], idxb.at[j])
    for j in range(3, 7):
        pltpu.async_copy(cre.at[wid, j], idxb.at[j], semi.at[j])
    for j in range(3):
        pltpu.async_copy(state.at[idxb.at[j, 0]], gbuf.at[j], semg.at[j])

    def _group(g, carry):
        i0 = g * _GRP
        for b10 in range(_GRP):
            _step(i0 + b10, b10)
        return carry

    lax.fori_loop(0, _NCHUNK // _GRP, _group, 0)

    for i in range(_NCHUNK - 2, _NCHUNK):
        pltpu.make_async_copy(gbuf.at[i % _NBUF], acc_sh.at[pl.ds(0, _CH)],
                              sems.at[i % _NBUF]).wait()
    plsc.subcore_barrier()

    def _out(k, carry):
        r0 = rbase + k * _RBLK
        pltpu.sync_copy(acc_sh.at[pl.ds(r0, _RBLK)],
                        acc.at[pl.ds(c * _N + r0, _RBLK)])
        return carry

    lax.fori_loop(0, nblk, _out, 0)


_sc_sparse = functools.partial(
    pl.kernel,
    out_type=jax.ShapeDtypeStruct((2 * _N, _U), jnp.float32),
    mesh=plsc.VectorSubcoreMesh(core_axis_name="c", subcore_axis_name="s",
                                num_cores=_NC, num_subcores=_NS),
    scratch_types=[
        pltpu.VMEM((_NBUF, _CH, _U), jnp.float32),
        pltpu.VMEM((_NIDX, 3, _CH), jnp.int32),
        pltpu.VMEM_SHARED((_N, _U), jnp.float32),
        pltpu.SemaphoreType.DMA((_NBUF,)),
        pltpu.SemaphoreType.DMA((_NBUF,)),
        pltpu.SemaphoreType.DMA((_NIDX,)),
    ],
)(_sc_body)


_BR = 2000


def _tc_body(a0, a1, wn, bn, gk, gb, out):
    a = a0[...] + a1[...]
    x = jnp.dot(a, wn[...], preferred_element_type=jnp.float32) + bn[...]
    mx = jnp.dot(x, gk[...], preferred_element_type=jnp.float32) + gb[0:1, :]
    b1 = gb[1:2, :]
    z = jax.nn.sigmoid(mx[:, :_U] + b1[:, :_U])
    r = jax.nn.sigmoid(mx[:, _U:2 * _U] + b1[:, _U:2 * _U])
    hh = jnp.tanh(mx[:, 2 * _U:] + r * b1[:, 2 * _U:])
    o = (1.0 - z) * hh
    ss = jnp.sum(o * o, axis=1, keepdims=True)
    out[...] = o * lax.rsqrt(jnp.maximum(ss, 1e-12))


def _tc_dense(acc, wn, bn2, gk, gb):
    nb = _N // _BR
    return pl.pallas_call(
        _tc_body,
        grid=(nb,),
        in_specs=[
            pl.BlockSpec((_BR, _U), lambda i: (i, 0)),
            pl.BlockSpec((_BR, _U), lambda i: (i + _N // _BR, 0)),
            pl.BlockSpec((_U, _U), lambda i: (0, 0)),
            pl.BlockSpec((1, _U), lambda i: (0, 0)),
            pl.BlockSpec((_U, 3 * _U), lambda i: (0, 0)),
            pl.BlockSpec((2, 3 * _U), lambda i: (0, 0)),
        ],
        out_specs=pl.BlockSpec((_BR, _U), lambda i: (i, 0)),
        out_shape=jax.ShapeDtypeStruct((_N, _U), jnp.float32),
        compiler_params=pltpu.CompilerParams(
            dimension_semantics=("arbitrary",),
        ),
    )(acc, acc, wn, bn2, gk, gb)


def kernel(edge_index, edge_values, message, Wn, bn, gru_kernel,
           gru_rec_kernel, gru_bias):
    del gru_rec_kernel
    pad = _EPAD - _E
    zpad = jnp.zeros((pad,), jnp.int32)
    row = jnp.concatenate(
        [edge_index[0].astype(jnp.int32), zpad]).reshape(_NW, _NCHUNK, _CH)
    col = jnp.concatenate(
        [edge_index[1].astype(jnp.int32), zpad]).reshape(_NW, _NCHUNK, _CH)
    evb = jnp.concatenate(
        [lax.bitcast_convert_type(edge_values.astype(jnp.float32),
                                  jnp.int32), zpad]).reshape(
                                      _NW, _NCHUNK, _CH)
    cre = jnp.stack([col, row, evb], axis=2)
    bn2 = bn.reshape(1, _U)

    state = message
    outs = []
    for _ in range(_REPS):
        acc = _sc_sparse(state, cre)
        state = _tc_dense(acc, Wn, bn2, gru_kernel, gru_bias)
        outs.append(state)

    out = jnp.concatenate(outs, axis=-1)
    return jnp.reshape(out, (_N, _U, _REPS))


# speedup vs baseline: 2.5887x; 2.5887x over previous
import functools

import jax
import jax.numpy as jnp
from jax import lax
from jax.experimental import pallas as pl
from jax.experimental.pallas import tpu as pltpu
from jax.experimental.pallas import tpu_sc as plsc

_N = 10000
_E = 320000
_U = 128
_REPS = 5

_NC = 2
_NS = 16
_NW = _NC * _NS
_CH = 64
_NCHUNK = 160
_EPW = _NCHUNK * _CH
_EPAD = _NW * _EPW
_NBUF = 5
_NIDX = 10
_GRP = 10
_RPS = 624
_RBLK = 16


def _sc_body(state, cre, acc, gbuf, idxb, acc_sh, semg, sems, semi):
    c = lax.axis_index("c")
    s = lax.axis_index("s")
    wid = s * _NC + c

    rbase = s * _RPS
    nblk = jnp.where(s == _NS - 1, (_N - (_NS - 1) * _RPS) // _RBLK,
                     _RPS // _RBLK)

    zv = jnp.zeros((16,), jnp.float32)
    for rr in range(_RBLK):
        for j in range(_U // 16):
            gbuf[0, rr, pl.ds(j * 16, 16)] = zv

    def _zero(k, carry):
        pltpu.sync_copy(gbuf.at[0, pl.ds(0, _RBLK)],
                        acc_sh.at[pl.ds(rbase + k * _RBLK, _RBLK)])
        return carry

    lax.fori_loop(0, nblk, _zero, 0)
    plsc.subcore_barrier()

    def _step(i, m):
        b = m % _NBUF
        pltpu.make_async_copy(state.at[pl.ds(0, _CH)], gbuf.at[b],
                              semg.at[b]).wait()

        def _scale(q, c2, b=b, m=m):
            ev16 = lax.bitcast_convert_type(
                idxb[m, 2, pl.ds(q * 16, 16)], jnp.float32)
            for l in range(16):
                sval = ev16[l]
                for j in range(_U // 16):
                    gbuf[b, q * 16 + l, pl.ds(j * 16, 16)] = (
                        gbuf[b, q * 16 + l, pl.ds(j * 16, 16)] * sval)
            return c2

        lax.fori_loop(0, _CH // 16, _scale, 0)

        pltpu.async_copy(gbuf.at[b], acc_sh.at[idxb.at[m, 1]], sems.at[b],
                         add=True)

        b3 = (b + 3) % _NBUF

        @pl.when(i >= 2)
        def _():
            pltpu.make_async_copy(gbuf.at[b3], acc_sh.at[pl.ds(0, _CH)],
                                  sems.at[b3]).wait()

        m3 = (m + 3) % _NIDX

        @pl.when(i + 3 < _NCHUNK)
        def _():
            pltpu.make_async_copy(cre.at[wid, 0], idxb.at[m3],
                                  semi.at[m3]).wait()
            pltpu.async_copy(state.at[idxb.at[m3, 0]], gbuf.at[b3],
                             semg.at[b3])

        m7 = (m + 7) % _NIDX

        @pl.when(i + 7 < _NCHUNK)
        def _():
            pltpu.async_copy(cre.at[wid, i + 7], idxb.at[m7], semi.at[m7])

    for j in range(3):
        pltpu.sync_copy(cre.at[wid, j], idxb.at[j])
    for j in range(3, 7):
        pltpu.async_copy(cre.at[wid, j], idxb.at[j], semi.at[j])
    for j in range(3):
        pltpu.async_copy(state.at[idxb.at[j, 0]], gbuf.at[j], semg.at[j])

    def _group(g, carry):
        i0 = g * _GRP
        for b10 in range(_GRP):
            _step(i0 + b10, b10)
        return carry

    lax.fori_loop(0, _NCHUNK // _GRP, _group, 0)

    for i in range(_NCHUNK - 2, _NCHUNK):
        pltpu.make_async_copy(gbuf.at[i % _NBUF], acc_sh.at[pl.ds(0, _CH)],
                              sems.at[i % _NBUF]).wait()
    plsc.subcore_barrier()

    def _out(k, carry):
        r0 = rbase + k * _RBLK
        pltpu.sync_copy(acc_sh.at[pl.ds(r0, _RBLK)],
                        acc.at[pl.ds(c * _N + r0, _RBLK)])
        return carry

    lax.fori_loop(0, nblk, _out, 0)


_sc_sparse = functools.partial(
    pl.kernel,
    out_type=jax.ShapeDtypeStruct((2 * _N, _U), jnp.float32),
    mesh=plsc.VectorSubcoreMesh(core_axis_name="c", subcore_axis_name="s",
                                num_cores=_NC, num_subcores=_NS),
    scratch_types=[
        pltpu.VMEM((_NBUF, _CH, _U), jnp.float32),
        pltpu.VMEM((_NIDX, 3, _CH), jnp.int32),
        pltpu.VMEM_SHARED((_N, _U), jnp.float32),
        pltpu.SemaphoreType.DMA((_NBUF,)),
        pltpu.SemaphoreType.DMA((_NBUF,)),
        pltpu.SemaphoreType.DMA((_NIDX,)),
    ],
)(_sc_body)


_BR = 2000


def _tc_body(a0, a1, wn, bn, gk, gb, out):
    a = a0[...] + a1[...]
    x = jnp.dot(a, wn[...], preferred_element_type=jnp.float32) + bn[...]
    mx = jnp.dot(x, gk[...], preferred_element_type=jnp.float32) + gb[0:1, :]
    b1 = gb[1:2, :]
    z = jax.nn.sigmoid(mx[:, :_U] + b1[:, :_U])
    r = jax.nn.sigmoid(mx[:, _U:2 * _U] + b1[:, _U:2 * _U])
    hh = jnp.tanh(mx[:, 2 * _U:] + r * b1[:, 2 * _U:])
    o = (1.0 - z) * hh
    ss = jnp.sum(o * o, axis=1, keepdims=True)
    out[...] = o * lax.rsqrt(jnp.maximum(ss, 1e-12))


def _tc_dense(acc, wn, bn2, gk, gb):
    nb = _N // _BR
    return pl.pallas_call(
        _tc_body,
        grid=(nb,),
        in_specs=[
            pl.BlockSpec((_BR, _U), lambda i: (i, 0)),
            pl.BlockSpec((_BR, _U), lambda i: (i + _N // _BR, 0)),
            pl.BlockSpec((_U, _U), lambda i: (0, 0)),
            pl.BlockSpec((1, _U), lambda i: (0, 0)),
            pl.BlockSpec((_U, 3 * _U), lambda i: (0, 0)),
            pl.BlockSpec((2, 3 * _U), lambda i: (0, 0)),
        ],
        out_specs=pl.BlockSpec((_BR, _U), lambda i: (i, 0)),
        out_shape=jax.ShapeDtypeStruct((_N, _U), jnp.float32),
        compiler_params=pltpu.CompilerParams(
            dimension_semantics=("arbitrary",),
        ),
    )(acc, acc, wn, bn2, gk, gb)


def kernel(edge_index, edge_values, message, Wn, bn, gru_kernel,
           gru_rec_kernel, gru_bias):
    del gru_rec_kernel
    pad = _EPAD - _E
    zpad = jnp.arange(pad, dtype=jnp.int32) % _N
    row = jnp.concatenate(
        [edge_index[0].astype(jnp.int32), zpad]).reshape(_NW, _NCHUNK, _CH)
    col = jnp.concatenate(
        [edge_index[1].astype(jnp.int32), zpad]).reshape(_NW, _NCHUNK, _CH)
    evb = jnp.concatenate(
        [lax.bitcast_convert_type(edge_values.astype(jnp.float32),
                                  jnp.int32), zpad]).reshape(
                                      _NW, _NCHUNK, _CH)
    cre = jnp.stack([col, row, evb], axis=2)
    bn2 = bn.reshape(1, _U)

    state = message
    outs = []
    for _ in range(_REPS):
        acc = _sc_sparse(state, cre)
        state = _tc_dense(acc, Wn, bn2, gru_kernel, gru_bias)
        outs.append(state)

    out = jnp.concatenate(outs, axis=-1)
    return jnp.reshape(out, (_N, _U, _REPS))


# speedup vs baseline: 2.7561x; 1.0647x over previous
import functools

import jax
import jax.numpy as jnp
from jax import lax
from jax.experimental import pallas as pl
from jax.experimental.pallas import tpu as pltpu
from jax.experimental.pallas import tpu_sc as plsc

_N = 10000
_E = 320000
_U = 128
_REPS = 5

_NC = 2
_NS = 16
_NW = _NC * _NS
_CH = 112
_NCHUNK = 90
_EPW = _NCHUNK * _CH
_EPAD = _NW * _EPW
_NBUF = 3
_NIDX = 6
_GRP = 6
_RPS = 624
_RBLK = 16


def _sc_body(state, cre, acc, gbuf, idxb, acc_sh, semg, sems, semi):
    c = lax.axis_index("c")
    s = lax.axis_index("s")
    wid = s * _NC + c

    rbase = s * _RPS
    nblk = jnp.where(s == _NS - 1, (_N - (_NS - 1) * _RPS) // _RBLK,
                     _RPS // _RBLK)

    zv = jnp.zeros((16,), jnp.float32)
    for rr in range(_RBLK):
        for j in range(_U // 16):
            gbuf[0, rr, pl.ds(j * 16, 16)] = zv

    def _zero(k, carry):
        pltpu.sync_copy(gbuf.at[0, pl.ds(0, _RBLK)],
                        acc_sh.at[pl.ds(rbase + k * _RBLK, _RBLK)])
        return carry

    lax.fori_loop(0, nblk, _zero, 0)
    plsc.subcore_barrier()

    def _step(i, m):
        b = m % _NBUF
        pltpu.make_async_copy(state.at[pl.ds(0, _CH)], gbuf.at[b],
                              semg.at[b]).wait()

        def _scale(q, c2, b=b, m=m):
            ev16 = lax.bitcast_convert_type(
                idxb[m, 2, pl.ds(q * 16, 16)], jnp.float32)
            for l in range(16):
                sval = ev16[l]
                for j in range(_U // 16):
                    gbuf[b, q * 16 + l, pl.ds(j * 16, 16)] = (
                        gbuf[b, q * 16 + l, pl.ds(j * 16, 16)] * sval)
            return c2

        lax.fori_loop(0, _CH // 16, _scale, 0)

        pltpu.async_copy(gbuf.at[b], acc_sh.at[idxb.at[m, 1]], sems.at[b],
                         add=True)

        b2 = (b + 2) % _NBUF

        @pl.when(i >= 1)
        def _():
            pltpu.make_async_copy(gbuf.at[b2], acc_sh.at[pl.ds(0, _CH)],
                                  sems.at[b2]).wait()

        m2 = (m + 2) % _NIDX

        @pl.when(i + 2 < _NCHUNK)
        def _():
            pltpu.make_async_copy(cre.at[wid, 0], idxb.at[m2],
                                  semi.at[m2]).wait()
            pltpu.async_copy(state.at[idxb.at[m2, 0]], gbuf.at[b2],
                             semg.at[b2])

        m5 = (m + 5) % _NIDX

        @pl.when(i + 5 < _NCHUNK)
        def _():
            pltpu.async_copy(cre.at[wid, i + 5], idxb.at[m5], semi.at[m5])

    for j in range(2):
        pltpu.sync_copy(cre.at[wid, j], idxb.at[j])
    for j in range(2, 5):
        pltpu.async_copy(cre.at[wid, j], idxb.at[j], semi.at[j])
    for j in range(2):
        pltpu.async_copy(state.at[idxb.at[j, 0]], gbuf.at[j], semg.at[j])

    def _group(g, carry):
        i0 = g * _GRP
        for bs in range(_GRP):
            _step(i0 + bs, bs)
        return carry

    lax.fori_loop(0, _NCHUNK // _GRP, _group, 0)

    for i in range(_NCHUNK - 1, _NCHUNK):
        pltpu.make_async_copy(gbuf.at[i % _NBUF], acc_sh.at[pl.ds(0, _CH)],
                              sems.at[i % _NBUF]).wait()
    plsc.subcore_barrier()

    def _out(k, carry):
        r0 = rbase + k * _RBLK
        pltpu.sync_copy(acc_sh.at[pl.ds(r0, _RBLK)],
                        acc.at[pl.ds(c * _N + r0, _RBLK)])
        return carry

    lax.fori_loop(0, nblk, _out, 0)


_sc_sparse = functools.partial(
    pl.kernel,
    out_type=jax.ShapeDtypeStruct((2 * _N, _U), jnp.float32),
    mesh=plsc.VectorSubcoreMesh(core_axis_name="c", subcore_axis_name="s",
                                num_cores=_NC, num_subcores=_NS),
    scratch_types=[
        pltpu.VMEM((_NBUF, _CH, _U), jnp.float32),
        pltpu.VMEM((_NIDX, 3, _CH), jnp.int32),
        pltpu.VMEM_SHARED((_N, _U), jnp.float32),
        pltpu.SemaphoreType.DMA((_NBUF,)),
        pltpu.SemaphoreType.DMA((_NBUF,)),
        pltpu.SemaphoreType.DMA((_NIDX,)),
    ],
)(_sc_body)


_BR = 2000


def _tc_body(a0, a1, wn, bn, gk, gb, out):
    a = a0[...] + a1[...]
    x = jnp.dot(a, wn[...], preferred_element_type=jnp.float32) + bn[...]
    mx = jnp.dot(x, gk[...], preferred_element_type=jnp.float32) + gb[0:1, :]
    b1 = gb[1:2, :]
    z = jax.nn.sigmoid(mx[:, :_U] + b1[:, :_U])
    r = jax.nn.sigmoid(mx[:, _U:2 * _U] + b1[:, _U:2 * _U])
    hh = jnp.tanh(mx[:, 2 * _U:] + r * b1[:, 2 * _U:])
    o = (1.0 - z) * hh
    ss = jnp.sum(o * o, axis=1, keepdims=True)
    out[...] = o * lax.rsqrt(jnp.maximum(ss, 1e-12))


def _tc_dense(acc, wn, bn2, gk, gb):
    nb = _N // _BR
    return pl.pallas_call(
        _tc_body,
        grid=(nb,),
        in_specs=[
            pl.BlockSpec((_BR, _U), lambda i: (i, 0)),
            pl.BlockSpec((_BR, _U), lambda i: (i + _N // _BR, 0)),
            pl.BlockSpec((_U, _U), lambda i: (0, 0)),
            pl.BlockSpec((1, _U), lambda i: (0, 0)),
            pl.BlockSpec((_U, 3 * _U), lambda i: (0, 0)),
            pl.BlockSpec((2, 3 * _U), lambda i: (0, 0)),
        ],
        out_specs=pl.BlockSpec((_BR, _U), lambda i: (i, 0)),
        out_shape=jax.ShapeDtypeStruct((_N, _U), jnp.float32),
        compiler_params=pltpu.CompilerParams(
            dimension_semantics=("arbitrary",),
        ),
    )(acc, acc, wn, bn2, gk, gb)


def kernel(edge_index, edge_values, message, Wn, bn, gru_kernel,
           gru_rec_kernel, gru_bias):
    del gru_rec_kernel
    pad = _EPAD - _E
    zpad = jnp.arange(pad, dtype=jnp.int32) % _N
    row = jnp.concatenate(
        [edge_index[0].astype(jnp.int32), zpad]).reshape(_NW, _NCHUNK, _CH)
    col = jnp.concatenate(
        [edge_index[1].astype(jnp.int32), zpad]).reshape(_NW, _NCHUNK, _CH)
    evb = jnp.concatenate(
        [lax.bitcast_convert_type(edge_values.astype(jnp.float32),
                                  jnp.int32), zpad]).reshape(
                                      _NW, _NCHUNK, _CH)
    cre = jnp.stack([col, row, evb], axis=2)
    bn2 = bn.reshape(1, _U)

    state = message
    outs = []
    for _ in range(_REPS):
        acc = _sc_sparse(state, cre)
        state = _tc_dense(acc, Wn, bn2, gru_kernel, gru_bias)
        outs.append(state)

    out = jnp.concatenate(outs, axis=-1)
    return jnp.reshape(out, (_N, _U, _REPS))


# speedup vs baseline: 3.1049x; 1.1266x over previous
import functools

import jax
import jax.numpy as jnp
from jax import lax
from jax.experimental import pallas as pl
from jax.experimental.pallas import tpu as pltpu
from jax.experimental.pallas import tpu_sc as plsc

_N = 10000
_E = 320000
_U = 128
_REPS = 5

_NC = 2
_NS = 16
_NW = _NC * _NS
_CH = 112
_NCHUNK = 90
_EPW = _NCHUNK * _CH
_EPAD = _NW * _EPW
_NBUF = 3
_NIDX = 6
_GRP = 6
_RPS = 624
_RBLK = 16


def _sc_body(state, cre, acc, gbuf, idxb, acc_sh, semg, sems, semi, semz):
    c = lax.axis_index("c")
    s = lax.axis_index("s")
    wid = s * _NC + c

    rbase = s * _RPS
    nblk = jnp.where(s == _NS - 1, (_N - (_NS - 1) * _RPS) // _RBLK,
                     _RPS // _RBLK)

    zv = jnp.zeros((16,), jnp.float32)
    for rr in range(_RBLK):
        for j in range(_U // 16):
            gbuf[0, rr, pl.ds(j * 16, 16)] = zv

    def _zero(k, carry):
        pltpu.async_copy(gbuf.at[0, pl.ds(0, _RBLK)],
                         acc_sh.at[pl.ds(rbase + k * _RBLK, _RBLK)], semz)
        return carry

    def _zwait(k, carry):
        pltpu.make_async_copy(gbuf.at[0, pl.ds(0, _RBLK)],
                              acc_sh.at[pl.ds(0, _RBLK)], semz).wait()
        return carry

    lax.fori_loop(0, nblk, _zero, 0)
    lax.fori_loop(0, nblk, _zwait, 0)
    plsc.subcore_barrier()

    def _step(i, m):
        b = m % _NBUF
        pltpu.make_async_copy(state.at[pl.ds(0, _CH)], gbuf.at[b],
                              semg.at[b]).wait()

        def _scale(q, c2, b=b, m=m):
            ev16 = lax.bitcast_convert_type(
                idxb[m, 2, pl.ds(q * 16, 16)], jnp.float32)
            for l in range(16):
                sval = ev16[l]
                for j in range(_U // 16):
                    gbuf[b, q * 16 + l, pl.ds(j * 16, 16)] = (
                        gbuf[b, q * 16 + l, pl.ds(j * 16, 16)] * sval)
            return c2

        lax.fori_loop(0, _CH // 16, _scale, 0)

        pltpu.async_copy(gbuf.at[b], acc_sh.at[idxb.at[m, 1]], sems.at[b],
                         add=True)

        b2 = (b + 2) % _NBUF

        @pl.when(i >= 1)
        def _():
            pltpu.make_async_copy(gbuf.at[b2], acc_sh.at[pl.ds(0, _CH)],
                                  sems.at[b2]).wait()

        m2 = (m + 2) % _NIDX

        @pl.when(i + 2 < _NCHUNK)
        def _():
            pltpu.make_async_copy(cre.at[wid, 0], idxb.at[m2],
                                  semi.at[m2]).wait()
            pltpu.async_copy(state.at[idxb.at[m2, 0]], gbuf.at[b2],
                             semg.at[b2])

        m5 = (m + 5) % _NIDX

        @pl.when(i + 5 < _NCHUNK)
        def _():
            pltpu.async_copy(cre.at[wid, i + 5], idxb.at[m5], semi.at[m5])

    for j in range(2):
        pltpu.sync_copy(cre.at[wid, j], idxb.at[j])
    for j in range(2, 5):
        pltpu.async_copy(cre.at[wid, j], idxb.at[j], semi.at[j])
    for j in range(2):
        pltpu.async_copy(state.at[idxb.at[j, 0]], gbuf.at[j], semg.at[j])

    def _group(g, carry):
        i0 = g * _GRP
        for bs in range(_GRP):
            _step(i0 + bs, bs)
        return carry

    lax.fori_loop(0, _NCHUNK // _GRP, _group, 0)

    for i in range(_NCHUNK - 1, _NCHUNK):
        pltpu.make_async_copy(gbuf.at[i % _NBUF], acc_sh.at[pl.ds(0, _CH)],
                              sems.at[i % _NBUF]).wait()
    plsc.subcore_barrier()

    @pl.when(s < _NS - 1)
    def _():
        pltpu.sync_copy(acc_sh.at[pl.ds(rbase, _RPS)],
                        acc.at[pl.ds(c * _N + rbase, _RPS)])

    @pl.when(s == _NS - 1)
    def _():
        last = _N - (_NS - 1) * _RPS
        pltpu.sync_copy(acc_sh.at[pl.ds((_NS - 1) * _RPS, last)],
                        acc.at[pl.ds(c * _N + (_NS - 1) * _RPS, last)])


_sc_sparse = functools.partial(
    pl.kernel,
    out_type=jax.ShapeDtypeStruct((2 * _N, _U), jnp.float32),
    mesh=plsc.VectorSubcoreMesh(core_axis_name="c", subcore_axis_name="s",
                                num_cores=_NC, num_subcores=_NS),
    scratch_types=[
        pltpu.VMEM((_NBUF, _CH, _U), jnp.float32),
        pltpu.VMEM((_NIDX, 3, _CH), jnp.int32),
        pltpu.VMEM_SHARED((_N, _U), jnp.float32),
        pltpu.SemaphoreType.DMA((_NBUF,)),
        pltpu.SemaphoreType.DMA((_NBUF,)),
        pltpu.SemaphoreType.DMA((_NIDX,)),
        pltpu.SemaphoreType.DMA,
    ],
)(_sc_body)


_BR = 2000


def _tc_body(a0, a1, wn, bn, gk, gb, out):
    a = a0[...] + a1[...]
    x = jnp.dot(a, wn[...], preferred_element_type=jnp.float32) + bn[...]
    mx = jnp.dot(x, gk[...], preferred_element_type=jnp.float32) + gb[0:1, :]
    b1 = gb[1:2, :]
    z = jax.nn.sigmoid(mx[:, :_U] + b1[:, :_U])
    r = jax.nn.sigmoid(mx[:, _U:2 * _U] + b1[:, _U:2 * _U])
    hh = jnp.tanh(mx[:, 2 * _U:] + r * b1[:, 2 * _U:])
    o = (1.0 - z) * hh
    ss = jnp.sum(o * o, axis=1, keepdims=True)
    out[...] = o * lax.rsqrt(jnp.maximum(ss, 1e-12))


def _tc_dense(acc, wn, bn2, gk, gb):
    nb = _N // _BR
    return pl.pallas_call(
        _tc_body,
        grid=(nb,),
        in_specs=[
            pl.BlockSpec((_BR, _U), lambda i: (i, 0)),
            pl.BlockSpec((_BR, _U), lambda i: (i + _N // _BR, 0)),
            pl.BlockSpec((_U, _U), lambda i: (0, 0)),
            pl.BlockSpec((1, _U), lambda i: (0, 0)),
            pl.BlockSpec((_U, 3 * _U), lambda i: (0, 0)),
            pl.BlockSpec((2, 3 * _U), lambda i: (0, 0)),
        ],
        out_specs=pl.BlockSpec((_BR, _U), lambda i: (i, 0)),
        out_shape=jax.ShapeDtypeStruct((_N, _U), jnp.float32),
        compiler_params=pltpu.CompilerParams(
            dimension_semantics=("arbitrary",),
        ),
    )(acc, acc, wn, bn2, gk, gb)


def kernel(edge_index, edge_values, message, Wn, bn, gru_kernel,
           gru_rec_kernel, gru_bias):
    del gru_rec_kernel
    pad = _EPAD - _E
    zpad = jnp.arange(pad, dtype=jnp.int32) % _N
    row = jnp.concatenate(
        [edge_index[0].astype(jnp.int32), zpad]).reshape(_NW, _NCHUNK, _CH)
    col = jnp.concatenate(
        [edge_index[1].astype(jnp.int32), zpad]).reshape(_NW, _NCHUNK, _CH)
    evb = jnp.concatenate(
        [lax.bitcast_convert_type(edge_values.astype(jnp.float32),
                                  jnp.int32), zpad]).reshape(
                                      _NW, _NCHUNK, _CH)
    cre = jnp.stack([col, row, evb], axis=2)
    bn2 = bn.reshape(1, _U)

    state = message
    outs = []
    for _ in range(_REPS):
        acc = _sc_sparse(state, cre)
        state = _tc_dense(acc, Wn, bn2, gru_kernel, gru_bias)
        outs.append(state)

    out = jnp.concatenate(outs, axis=-1)
    return jnp.reshape(out, (_N, _U, _REPS))


# speedup vs baseline: 3.1459x; 1.0132x over previous
import functools

import jax
import jax.numpy as jnp
from jax import lax
from jax.experimental import pallas as pl
from jax.experimental.pallas import tpu as pltpu
from jax.experimental.pallas import tpu_sc as plsc

_N = 10000
_E = 320000
_U = 128
_REPS = 5

_NC = 2
_NS = 16
_NW = _NC * _NS
_CH = 112
_NCHUNK = 90
_EPW = _NCHUNK * _CH
_EPAD = _NW * _EPW
_NBUF = 3
_NIDX = 6
_GRP = 6
_RPS = 624
_RBLK = 16


def _sc_body(state, cre, acc, gbuf, idxb, acc_sh, semg, sems, semi, semz):
    c = lax.axis_index("c")
    s = lax.axis_index("s")
    wid = s * _NC + c

    rbase = s * _RPS
    nblk = jnp.where(s == _NS - 1, (_N - (_NS - 1) * _RPS) // _RBLK,
                     _RPS // _RBLK)


    def _step(i, m):
        b = m % _NBUF
        pltpu.make_async_copy(state.at[pl.ds(0, _CH)], gbuf.at[b],
                              semg.at[b]).wait()

        def _scale(q, c2, b=b, m=m):
            ev16 = lax.bitcast_convert_type(
                idxb[m, 2, pl.ds(q * 16, 16)], jnp.float32)
            for l in range(16):
                sval = ev16[l]
                for j in range(_U // 16):
                    gbuf[b, q * 16 + l, pl.ds(j * 16, 16)] = (
                        gbuf[b, q * 16 + l, pl.ds(j * 16, 16)] * sval)
            return c2

        lax.fori_loop(0, _CH // 16, _scale, 0)

        pltpu.async_copy(gbuf.at[b], acc_sh.at[idxb.at[m, 1]], sems.at[b],
                         add=True)

        b2 = (b + 2) % _NBUF

        @pl.when(i >= 1)
        def _():
            pltpu.make_async_copy(gbuf.at[b2], acc_sh.at[pl.ds(0, _CH)],
                                  sems.at[b2]).wait()

        m2 = (m + 2) % _NIDX

        @pl.when(i + 2 < _NCHUNK)
        def _():
            pltpu.make_async_copy(cre.at[wid, 0], idxb.at[m2],
                                  semi.at[m2]).wait()
            pltpu.async_copy(state.at[idxb.at[m2, 0]], gbuf.at[b2],
                             semg.at[b2])

        m5 = (m + 5) % _NIDX

        @pl.when(i + 5 < _NCHUNK)
        def _():
            pltpu.async_copy(cre.at[wid, i + 5], idxb.at[m5], semi.at[m5])

    for j in range(2):
        pltpu.sync_copy(cre.at[wid, j], idxb.at[j])
    for j in range(2, 5):
        pltpu.async_copy(cre.at[wid, j], idxb.at[j], semi.at[j])
    for j in range(2):
        pltpu.async_copy(state.at[idxb.at[j, 0]], gbuf.at[j], semg.at[j])

    zv = jnp.zeros((16,), jnp.float32)
    for rr in range(_RBLK):
        for j in range(_U // 16):
            gbuf[2, rr, pl.ds(j * 16, 16)] = zv

    def _zero(k, carry):
        pltpu.async_copy(gbuf.at[2, pl.ds(0, _RBLK)],
                         acc_sh.at[pl.ds(rbase + k * _RBLK, _RBLK)], semz)
        return carry

    def _zwait(k, carry):
        pltpu.make_async_copy(gbuf.at[2, pl.ds(0, _RBLK)],
                              acc_sh.at[pl.ds(0, _RBLK)], semz).wait()
        return carry

    lax.fori_loop(0, nblk, _zero, 0)
    lax.fori_loop(0, nblk, _zwait, 0)
    plsc.subcore_barrier()

    def _group(g, carry):
        i0 = g * _GRP
        for bs in range(_GRP):
            _step(i0 + bs, bs)
        return carry

    lax.fori_loop(0, _NCHUNK // _GRP, _group, 0)

    for i in range(_NCHUNK - 1, _NCHUNK):
        pltpu.make_async_copy(gbuf.at[i % _NBUF], acc_sh.at[pl.ds(0, _CH)],
                              sems.at[i % _NBUF]).wait()
    plsc.subcore_barrier()

    @pl.when(s < _NS - 1)
    def _():
        pltpu.sync_copy(acc_sh.at[pl.ds(rbase, _RPS)],
                        acc.at[pl.ds(c * _N + rbase, _RPS)])

    @pl.when(s == _NS - 1)
    def _():
        last = _N - (_NS - 1) * _RPS
        pltpu.sync_copy(acc_sh.at[pl.ds((_NS - 1) * _RPS, last)],
                        acc.at[pl.ds(c * _N + (_NS - 1) * _RPS, last)])


_sc_sparse = functools.partial(
    pl.kernel,
    out_type=jax.ShapeDtypeStruct((2 * _N, _U), jnp.float32),
    mesh=plsc.VectorSubcoreMesh(core_axis_name="c", subcore_axis_name="s",
                                num_cores=_NC, num_subcores=_NS),
    scratch_types=[
        pltpu.VMEM((_NBUF, _CH, _U), jnp.float32),
        pltpu.VMEM((_NIDX, 3, _CH), jnp.int32),
        pltpu.VMEM_SHARED((_N, _U), jnp.float32),
        pltpu.SemaphoreType.DMA((_NBUF,)),
        pltpu.SemaphoreType.DMA((_NBUF,)),
        pltpu.SemaphoreType.DMA((_NIDX,)),
        pltpu.SemaphoreType.DMA,
    ],
)(_sc_body)


_BR = 5000


def _tc_body(a0, a1, wn, bn, gk, gb, out):
    a = a0[...] + a1[...]
    x = jnp.dot(a, wn[...], preferred_element_type=jnp.float32) + bn[...]
    mx = jnp.dot(x, gk[...], preferred_element_type=jnp.float32) + gb[0:1, :]
    b1 = gb[1:2, :]
    z = jax.nn.sigmoid(mx[:, :_U] + b1[:, :_U])
    r = jax.nn.sigmoid(mx[:, _U:2 * _U] + b1[:, _U:2 * _U])
    hh = jnp.tanh(mx[:, 2 * _U:] + r * b1[:, 2 * _U:])
    o = (1.0 - z) * hh
    ss = jnp.sum(o * o, axis=1, keepdims=True)
    out[...] = o * lax.rsqrt(jnp.maximum(ss, 1e-12))


def _tc_dense(acc, wn, bn2, gk, gb):
    nb = _N // _BR
    return pl.pallas_call(
        _tc_body,
        grid=(nb,),
        in_specs=[
            pl.BlockSpec((_BR, _U), lambda i: (i, 0)),
            pl.BlockSpec((_BR, _U), lambda i: (i + _N // _BR, 0)),
            pl.BlockSpec((_U, _U), lambda i: (0, 0)),
            pl.BlockSpec((1, _U), lambda i: (0, 0)),
            pl.BlockSpec((_U, 3 * _U), lambda i: (0, 0)),
            pl.BlockSpec((2, 3 * _U), lambda i: (0, 0)),
        ],
        out_specs=pl.BlockSpec((_BR, _U), lambda i: (i, 0)),
        out_shape=jax.ShapeDtypeStruct((_N, _U), jnp.float32),
        compiler_params=pltpu.CompilerParams(
            dimension_semantics=("arbitrary",),
        ),
    )(acc, acc, wn, bn2, gk, gb)


def kernel(edge_index, edge_values, message, Wn, bn, gru_kernel,
           gru_rec_kernel, gru_bias):
    del gru_rec_kernel
    pad = _EPAD - _E
    zpad = jnp.arange(pad, dtype=jnp.int32) % _N
    row = jnp.concatenate(
        [edge_index[0].astype(jnp.int32), zpad]).reshape(_NW, _NCHUNK, _CH)
    col = jnp.concatenate(
        [edge_index[1].astype(jnp.int32), zpad]).reshape(_NW, _NCHUNK, _CH)
    evb = jnp.concatenate(
        [lax.bitcast_convert_type(edge_values.astype(jnp.float32),
                                  jnp.int32), zpad]).reshape(
                                      _NW, _NCHUNK, _CH)
    cre = jnp.stack([col, row, evb], axis=2)
    bn2 = bn.reshape(1, _U)

    state = message
    outs = []
    for _ in range(_REPS):
        acc = _sc_sparse(state, cre)
        state = _tc_dense(acc, Wn, bn2, gru_kernel, gru_bias)
        outs.append(state)

    out = jnp.concatenate(outs, axis=-1)
    return jnp.reshape(out, (_N, _U, _REPS))
